# baseline (speedup 1.0000x reference)
import functools

import numpy as np
import jax
import jax.numpy as jnp
from jax import lax
from jax.experimental import pallas as pl
from jax.experimental.pallas import tpu as pltpu
from jax.experimental.pallas import tpu_sc as plsc

_B, _IN_CH, _T = 32, 4, 16384
_OUT_CH, _K = 32, 1023
_NE, _ED = 1024, 1023
_EDP = 1024
_COMMIT = 0.25
_NROWS = _B * _OUT_CH * _IN_CH
_BM = 256
_HALF = _K // 2

_PERM = np.empty(_NROWS, np.int32)
for _b in range(_B):
    for _oc in range(_OUT_CH):
        for _ic in range(_IN_CH):
            _rf = (_b * _OUT_CH + _oc) * _IN_CH + _ic
            _rn = ((((_b + _B // 2) % _B) * _OUT_CH + (_oc + _OUT_CH // 2) % _OUT_CH)
                   * _IN_CH + (_ic + _IN_CH // 2) % _IN_CH)
            _PERM[_rf] = _rn
_PERM.setflags(write=False)


def _extract(X):
    t = X.shape[-1]
    padded = int(np.ceil(t / _K) * _K)
    end = padded - _K - 1 - _K
    positions = jnp.linspace(0.0, float(end), _OUT_CH).astype(jnp.int32)
    idx = positions[:, None] + jnp.arange(_K, dtype=jnp.int32)[None, :]
    filt = X[:, :, idx]
    return jnp.transpose(filt, (0, 2, 1, 3))


def _acov_preshift(f):
    eps = jnp.finfo(f.dtype).eps
    n = f.shape[-1]
    fmax = jnp.max(jnp.abs(f), axis=-1, keepdims=True)
    fmax = jnp.where(fmax == 0, eps, fmax)
    win = 0.5 * (1.0 - jnp.cos(2.0 * jnp.pi * jnp.arange(n, dtype=f.dtype) / n))
    wd = f * win / fmax
    spec = jnp.fft.rfft(wd, n=n) ** 2
    return jnp.fft.irfft(jnp.abs(spec), n=n).astype(f.dtype)


def _vq_kernel(f_ref, w_ref, idx_ref, loss_ref, perp_ref, cnt_ref, dsum_ref):
    i = pl.program_id(0)
    f_raw = f_ref[...]
    f = jnp.concatenate([f_raw[:, _HALF:], f_raw[:, :_HALF]], axis=1)
    w = w_ref[...]
    a = jnp.sum(f * f, axis=1, keepdims=True)
    b = jnp.sum(w * w, axis=1)
    mm = jax.lax.dot_general(f, w, (((1,), (1,)), ((), ())),
                             preferred_element_type=jnp.float32)
    d = (a + b[None, :]) - 2.0 * mm
    m = jnp.min(d, axis=1, keepdims=True)
    jcol = jax.lax.broadcasted_iota(jnp.int32, d.shape, 1)
    idx = jnp.min(jnp.where(d == m, jcol, jnp.int32(2 ** 30)), axis=1)
    idx_ref[...] = idx[:, None]

    oh = jnp.where(jcol == idx[:, None], 1.0, 0.0).astype(jnp.float32)
    blk_cnt = jnp.sum(oh, axis=0, keepdims=True)
    blk_dsum = jnp.reshape(jnp.sum(m), (1, 1))

    @pl.when(i == 0)
    def _():
        cnt_ref[...] = blk_cnt
        dsum_ref[...] = blk_dsum

    @pl.when(i > 0)
    def _():
        cnt_ref[...] = cnt_ref[...] + blk_cnt
        dsum_ref[...] = dsum_ref[...] + blk_dsum

    @pl.when(i == (_NROWS // _BM) - 1)
    def _():
        mean_d = dsum_ref[...] / jnp.float32(_NROWS * _ED)
        loss_ref[...] = mean_d + _COMMIT * mean_d
        p = cnt_ref[...] / jnp.float32(_NROWS)
        feps = jnp.finfo(jnp.float32).eps
        ent = -jnp.sum(p * jnp.log(p + feps))
        perp_ref[...] = jnp.reshape(jnp.exp(ent), (1, 1))


_NC, _NS = 2, 16
_NW = _NC * _NS
_BPW = _NROWS // _NW
_CHUNK = 64


def _sc_gather(Wpad, idx):
    mesh = plsc.VectorSubcoreMesh(core_axis_name="c", subcore_axis_name="s")

    @functools.partial(
        pl.kernel, mesh=mesh,
        out_type=jax.ShapeDtypeStruct((_NROWS, _EDP), jnp.float32),
        scratch_types=[pltpu.VMEM((_CHUNK,), jnp.int32),
                       pltpu.VMEM((_CHUNK, _EDP), jnp.float32),
                       pltpu.SemaphoreType.DMA],
    )
    def k(table_hbm, idx_hbm, out_hbm, idx_v, rows_v, sem):
        wid = lax.axis_index("s") * _NC + lax.axis_index("c")
        base = wid * _BPW
        for c in range(_BPW // _CHUNK):
            off = base + c * _CHUNK
            pltpu.sync_copy(idx_hbm.at[pl.ds(off, _CHUNK)], idx_v)
            pltpu.async_copy(table_hbm.at[idx_v], rows_v, sem).wait()
            pltpu.sync_copy(rows_v, out_hbm.at[pl.ds(off, _CHUNK)])

    return k(Wpad, idx)


def kernel(X, W):
    acov = _acov_preshift(_extract(X))
    flat_pre = acov.reshape(_NROWS, _ED)

    nblk = _NROWS // _BM
    idx2d, loss, perp = pl.pallas_call(
        _vq_kernel,
        grid=(nblk,),
        in_specs=[pl.BlockSpec((_BM, _ED), lambda i: (i, 0)),
                  pl.BlockSpec((_NE, _ED), lambda i: (0, 0))],
        out_specs=[pl.BlockSpec((_BM, 1), lambda i: (i, 0)),
                   pl.BlockSpec((1, 1), lambda i: (0, 0)),
                   pl.BlockSpec((1, 1), lambda i: (0, 0))],
        out_shape=[jax.ShapeDtypeStruct((_NROWS, 1), jnp.int32),
                   jax.ShapeDtypeStruct((1, 1), jnp.float32),
                   jax.ShapeDtypeStruct((1, 1), jnp.float32)],
        scratch_shapes=[pltpu.VMEM((1, _NE), jnp.float32),
                        pltpu.VMEM((1, 1), jnp.float32)],
        compiler_params=pltpu.CompilerParams(
            dimension_semantics=("arbitrary",)),
    )(flat_pre, W)

    idx_final = idx2d[:, 0][jnp.asarray(_PERM)]
    Wpad = jnp.concatenate(
        [W, jnp.zeros((_NE, _EDP - _ED), jnp.float32)], axis=1)
    q_pad = _sc_gather(Wpad, idx_final)
    q = q_pad[:, :_ED].reshape(_B, _OUT_CH, _IN_CH, _K)
    return loss[0, 0], q, perp[0, 0]


# speedup vs baseline: 1.0212x; 1.0212x over previous
import functools

import numpy as np
import jax
import jax.numpy as jnp
from jax import lax
from jax.experimental import pallas as pl
from jax.experimental.pallas import tpu as pltpu
from jax.experimental.pallas import tpu_sc as plsc

_B, _IN_CH, _T = 32, 4, 16384
_OUT_CH, _K = 32, 1023
_NE, _ED = 1024, 1023
_EDP = 1024
_COMMIT = 0.25
_NROWS = _B * _OUT_CH * _IN_CH
_BM = 256
_HALF = _K // 2

_PERM = np.empty(_NROWS, np.int32)
for _b in range(_B):
    for _oc in range(_OUT_CH):
        for _ic in range(_IN_CH):
            _rf = (_b * _OUT_CH + _oc) * _IN_CH + _ic
            _rn = ((((_b + _B // 2) % _B) * _IN_CH + (_ic + _IN_CH // 2) % _IN_CH)
                   * _OUT_CH + (_oc + _OUT_CH // 2) % _OUT_CH)
            _PERM[_rf] = _rn
_PERM.setflags(write=False)


def _extract(X):
    t = X.shape[-1]
    padded = int(np.ceil(t / _K) * _K)
    end = padded - _K - 1 - _K
    positions = jnp.linspace(0.0, float(end), _OUT_CH).astype(jnp.int32)
    idx = positions[:, None] + jnp.arange(_K, dtype=jnp.int32)[None, :]
    return X[:, :, idx]


def _acov_preshift(f):
    eps = jnp.finfo(f.dtype).eps
    n = f.shape[-1]
    fmax = jnp.max(jnp.abs(f), axis=-1, keepdims=True)
    fmax = jnp.where(fmax == 0, eps, fmax)
    win = 0.5 * (1.0 - jnp.cos(2.0 * jnp.pi * jnp.arange(n, dtype=f.dtype) / n))
    wd = f * win / fmax
    spec = jnp.fft.rfft(wd, n=n) ** 2
    return jnp.fft.irfft(jnp.abs(spec), n=n).astype(f.dtype)


def _vq_kernel(f_ref, w_ref, idx_ref, loss_ref, perp_ref, cnt_ref, dsum_ref):
    i = pl.program_id(0)
    f_raw = f_ref[...]
    f = jnp.concatenate([f_raw[:, _HALF:], f_raw[:, :_HALF]], axis=1)
    w = w_ref[...]
    a = jnp.sum(f * f, axis=1, keepdims=True)
    b = jnp.sum(w * w, axis=1)
    mm = jax.lax.dot_general(f, w, (((1,), (1,)), ((), ())),
                             preferred_element_type=jnp.float32)
    d = (a + b[None, :]) - 2.0 * mm
    m = jnp.min(d, axis=1, keepdims=True)
    jcol = jax.lax.broadcasted_iota(jnp.int32, d.shape, 1)
    idx = jnp.min(jnp.where(d == m, jcol, jnp.int32(2 ** 30)), axis=1)
    idx_ref[...] = idx[:, None]

    oh = jnp.where(jcol == idx[:, None], 1.0, 0.0).astype(jnp.float32)
    blk_cnt = jnp.sum(oh, axis=0, keepdims=True)
    blk_dsum = jnp.reshape(jnp.sum(m), (1, 1))

    @pl.when(i == 0)
    def _():
        cnt_ref[...] = blk_cnt
        dsum_ref[...] = blk_dsum

    @pl.when(i > 0)
    def _():
        cnt_ref[...] = cnt_ref[...] + blk_cnt
        dsum_ref[...] = dsum_ref[...] + blk_dsum

    @pl.when(i == (_NROWS // _BM) - 1)
    def _():
        mean_d = dsum_ref[...] / jnp.float32(_NROWS * _ED)
        loss_ref[...] = mean_d + _COMMIT * mean_d
        p = cnt_ref[...] / jnp.float32(_NROWS)
        feps = jnp.finfo(jnp.float32).eps
        ent = -jnp.sum(p * jnp.log(p + feps))
        perp_ref[...] = jnp.reshape(jnp.exp(ent), (1, 1))


_NC, _NS = 2, 16
_NW = _NC * _NS
_BPW = _NROWS // _NW
_CHUNK = 64


def _sc_gather(Wpad, idx):
    mesh = plsc.VectorSubcoreMesh(core_axis_name="c", subcore_axis_name="s")

    @functools.partial(
        pl.kernel, mesh=mesh,
        out_type=jax.ShapeDtypeStruct((_NROWS, _EDP), jnp.float32),
        scratch_types=[pltpu.VMEM((_CHUNK,), jnp.int32),
                       pltpu.VMEM((_CHUNK, _EDP), jnp.float32),
                       pltpu.SemaphoreType.DMA],
    )
    def k(table_hbm, idx_hbm, out_hbm, idx_v, rows_v, sem):
        wid = lax.axis_index("s") * _NC + lax.axis_index("c")
        base = wid * _BPW
        for c in range(_BPW // _CHUNK):
            off = base + c * _CHUNK
            pltpu.sync_copy(idx_hbm.at[pl.ds(off, _CHUNK)], idx_v)
            pltpu.async_copy(table_hbm.at[idx_v], rows_v, sem).wait()
            pltpu.sync_copy(rows_v, out_hbm.at[pl.ds(off, _CHUNK)])

    return k(Wpad, idx)


def _unpad_kernel(qp_ref, q_ref):
    q_ref[...] = qp_ref[:, :_ED]


def kernel(X, W):
    acov = _acov_preshift(_extract(X))
    flat_pre = acov.reshape(_NROWS, _ED)

    nblk = _NROWS // _BM
    idx2d, loss, perp = pl.pallas_call(
        _vq_kernel,
        grid=(nblk,),
        in_specs=[pl.BlockSpec((_BM, _ED), lambda i: (i, 0)),
                  pl.BlockSpec((_NE, _ED), lambda i: (0, 0))],
        out_specs=[pl.BlockSpec((_BM, 1), lambda i: (i, 0)),
                   pl.BlockSpec((1, 1), lambda i: (0, 0)),
                   pl.BlockSpec((1, 1), lambda i: (0, 0))],
        out_shape=[jax.ShapeDtypeStruct((_NROWS, 1), jnp.int32),
                   jax.ShapeDtypeStruct((1, 1), jnp.float32),
                   jax.ShapeDtypeStruct((1, 1), jnp.float32)],
        scratch_shapes=[pltpu.VMEM((1, _NE), jnp.float32),
                        pltpu.VMEM((1, 1), jnp.float32)],
        compiler_params=pltpu.CompilerParams(
            dimension_semantics=("arbitrary",)),
    )(flat_pre, W)

    idx_final = idx2d[:, 0][jnp.asarray(_PERM)]
    Wpad = jnp.concatenate(
        [W, jnp.zeros((_NE, _EDP - _ED), jnp.float32)], axis=1)
    q_pad = _sc_gather(Wpad, idx_final)
    q = pl.pallas_call(
        _unpad_kernel,
        grid=(_NROWS // 512,),
        in_specs=[pl.BlockSpec((512, _EDP), lambda i: (i, 0))],
        out_specs=pl.BlockSpec((512, _ED), lambda i: (i, 0)),
        out_shape=jax.ShapeDtypeStruct((_NROWS, _ED), jnp.float32),
        compiler_params=pltpu.CompilerParams(
            dimension_semantics=("arbitrary",)),
    )(q_pad)
    return loss[0, 0], q.reshape(_B, _OUT_CH, _IN_CH, _K), perp[0, 0]


# speedup vs baseline: 1.0278x; 1.0065x over previous
import functools

import numpy as np
import jax
import jax.numpy as jnp
from jax import lax
from jax.experimental import pallas as pl
from jax.experimental.pallas import tpu as pltpu
from jax.experimental.pallas import tpu_sc as plsc

_B, _IN_CH, _T = 32, 4, 16384
_OUT_CH, _K = 32, 1023
_NE, _ED = 1024, 1023
_EDP = 1024
_COMMIT = 0.25
_NROWS = _B * _OUT_CH * _IN_CH
_BM = 512
_HALF = _K // 2

_PERM = np.empty(_NROWS, np.int32)
for _b in range(_B):
    for _oc in range(_OUT_CH):
        for _ic in range(_IN_CH):
            _rf = (_b * _OUT_CH + _oc) * _IN_CH + _ic
            _rn = ((((_b + _B // 2) % _B) * _IN_CH + (_ic + _IN_CH // 2) % _IN_CH)
                   * _OUT_CH + (_oc + _OUT_CH // 2) % _OUT_CH)
            _PERM[_rf] = _rn
_PERM.setflags(write=False)


def _extract(X):
    t = X.shape[-1]
    padded = int(np.ceil(t / _K) * _K)
    end = padded - _K - 1 - _K
    positions = jnp.linspace(0.0, float(end), _OUT_CH).astype(jnp.int32)
    idx = positions[:, None] + jnp.arange(_K, dtype=jnp.int32)[None, :]
    return X[:, :, idx]


def _acov_preshift(f):
    eps = jnp.finfo(f.dtype).eps
    n = f.shape[-1]
    fmax = jnp.max(jnp.abs(f), axis=-1, keepdims=True)
    fmax = jnp.where(fmax == 0, eps, fmax)
    win = 0.5 * (1.0 - jnp.cos(2.0 * jnp.pi * jnp.arange(n, dtype=f.dtype) / n))
    wd = f * win / fmax
    spec = jnp.fft.rfft(wd, n=n) ** 2
    return jnp.fft.irfft(jnp.abs(spec), n=n).astype(f.dtype)


def _vq_kernel(f_ref, w_ref, idx_ref, loss_ref, perp_ref, cnt_ref, dsum_ref):
    i = pl.program_id(0)
    f_raw = f_ref[...]
    f = jnp.concatenate([f_raw[:, _HALF:], f_raw[:, :_HALF]], axis=1)
    w = w_ref[...]
    a = jnp.sum(f * f, axis=1, keepdims=True)
    b = jnp.sum(w * w, axis=1)
    mm = jax.lax.dot_general(f, w, (((1,), (1,)), ((), ())),
                             preferred_element_type=jnp.float32)
    d = (a + b[None, :]) - 2.0 * mm
    m = jnp.min(d, axis=1, keepdims=True)
    jcol = jax.lax.broadcasted_iota(jnp.int32, d.shape, 1)
    idx = jnp.min(jnp.where(d == m, jcol, jnp.int32(2 ** 30)), axis=1)
    idx_ref[...] = idx[:, None]

    oh = jnp.where(jcol == idx[:, None], 1.0, 0.0).astype(jnp.float32)
    blk_cnt = jnp.sum(oh, axis=0, keepdims=True)
    blk_dsum = jnp.reshape(jnp.sum(m), (1, 1))

    @pl.when(i == 0)
    def _():
        cnt_ref[...] = blk_cnt
        dsum_ref[...] = blk_dsum

    @pl.when(i > 0)
    def _():
        cnt_ref[...] = cnt_ref[...] + blk_cnt
        dsum_ref[...] = dsum_ref[...] + blk_dsum

    @pl.when(i == (_NROWS // _BM) - 1)
    def _():
        mean_d = dsum_ref[...] / jnp.float32(_NROWS * _ED)
        loss_ref[...] = mean_d + _COMMIT * mean_d
        p = cnt_ref[...] / jnp.float32(_NROWS)
        feps = jnp.finfo(jnp.float32).eps
        ent = -jnp.sum(p * jnp.log(p + feps))
        perp_ref[...] = jnp.reshape(jnp.exp(ent), (1, 1))


_NC, _NS = 2, 16
_NW = _NC * _NS
_BPW = _NROWS // _NW
_CHUNK = 64


def _sc_gather(Wpad, idx):
    mesh = plsc.VectorSubcoreMesh(core_axis_name="c", subcore_axis_name="s")

    @functools.partial(
        pl.kernel, mesh=mesh,
        out_type=jax.ShapeDtypeStruct((_NROWS, _EDP), jnp.float32),
        scratch_types=[pltpu.VMEM((_CHUNK,), jnp.int32),
                       pltpu.VMEM((_CHUNK, _EDP), jnp.float32),
                       pltpu.SemaphoreType.DMA],
    )
    def k(table_hbm, idx_hbm, out_hbm, idx_v, rows_v, sem):
        wid = lax.axis_index("s") * _NC + lax.axis_index("c")
        base = wid * _BPW
        for c in range(_BPW // _CHUNK):
            off = base + c * _CHUNK
            pltpu.sync_copy(idx_hbm.at[pl.ds(off, _CHUNK)], idx_v)
            pltpu.async_copy(table_hbm.at[idx_v], rows_v, sem).wait()
            pltpu.sync_copy(rows_v, out_hbm.at[pl.ds(off, _CHUNK)])

    return k(Wpad, idx)


def _unpad_kernel(qp_ref, q_ref):
    q_ref[...] = qp_ref[:, :_ED]


def kernel(X, W):
    acov = _acov_preshift(_extract(X))
    flat_pre = acov.reshape(_NROWS, _ED)

    nblk = _NROWS // _BM
    idx2d, loss, perp = pl.pallas_call(
        _vq_kernel,
        grid=(nblk,),
        in_specs=[pl.BlockSpec((_BM, _ED), lambda i: (i, 0)),
                  pl.BlockSpec((_NE, _ED), lambda i: (0, 0))],
        out_specs=[pl.BlockSpec((_BM, 1), lambda i: (i, 0)),
                   pl.BlockSpec((1, 1), lambda i: (0, 0)),
                   pl.BlockSpec((1, 1), lambda i: (0, 0))],
        out_shape=[jax.ShapeDtypeStruct((_NROWS, 1), jnp.int32),
                   jax.ShapeDtypeStruct((1, 1), jnp.float32),
                   jax.ShapeDtypeStruct((1, 1), jnp.float32)],
        scratch_shapes=[pltpu.VMEM((1, _NE), jnp.float32),
                        pltpu.VMEM((1, 1), jnp.float32)],
        compiler_params=pltpu.CompilerParams(
            dimension_semantics=("arbitrary",)),
    )(flat_pre, W)

    idx_final = idx2d[:, 0][jnp.asarray(_PERM)]
    Wpad = jnp.concatenate(
        [W, jnp.zeros((_NE, _EDP - _ED), jnp.float32)], axis=1)
    q_pad = _sc_gather(Wpad, idx_final)
    q = pl.pallas_call(
        _unpad_kernel,
        grid=(_NROWS // 512,),
        in_specs=[pl.BlockSpec((512, _EDP), lambda i: (i, 0))],
        out_specs=pl.BlockSpec((512, _ED), lambda i: (i, 0)),
        out_shape=jax.ShapeDtypeStruct((_NROWS, _ED), jnp.float32),
        compiler_params=pltpu.CompilerParams(
            dimension_semantics=("arbitrary",)),
    )(q_pad)
    return loss[0, 0], q.reshape(_B, _OUT_CH, _IN_CH, _K), perp[0, 0]
